# Initial kernel scaffold; baseline (speedup 1.0000x reference)
import jax
import jax.numpy as jnp
from jax.experimental import pallas as pl


def kernel(features, features_mask, codebook):
    raise NotImplementedError("write your pallas kernel here")


# speedup vs baseline: 1.0029x; 1.0029x over previous
import jax
import jax.numpy as jnp
from jax import lax
from jax.experimental import pallas as pl


def _vq_body(x_ref, m_ref, c_ref, q_ref, sq_ref):
    x = x_ref[0]
    c = c_ref[...]
    n = c.shape[0]
    xn = jnp.sum(x * x, axis=1, keepdims=True)
    cn = jnp.sum(c * c, axis=1)
    xc = lax.dot_general(x, c, (((1,), (1,)), ((), ())),
                         preferred_element_type=jnp.float32,
                         precision=lax.Precision.DEFAULT)
    dist = xn - 2.0 * xc + cn[None, :]
    mind = jnp.min(dist, axis=1, keepdims=True)
    ids = lax.broadcasted_iota(jnp.int32, dist.shape, 1)
    idx = jnp.min(jnp.where(dist == mind, ids, n), axis=1)
    oh = (ids == idx[:, None]).astype(jnp.float32)
    q = lax.dot_general(oh, c, (((1,), (0,)), ((), ())),
                        preferred_element_type=jnp.float32,
                        precision=lax.Precision.HIGHEST)
    sq = jnp.sum((x - q) ** 2, axis=1)
    q_ref[0] = q
    sq_ref[0, 0] = sq * m_ref[0, 0]


def kernel(features, features_mask, codebook):
    B, T, H = features.shape
    N = codebook.shape[0]
    mask3 = features_mask.reshape(B, 1, T)
    q, sqm = pl.pallas_call(
        _vq_body,
        grid=(B,),
        in_specs=[
            pl.BlockSpec((1, T, H), lambda i: (i, 0, 0)),
            pl.BlockSpec((1, 1, T), lambda i: (i, 0, 0)),
            pl.BlockSpec((N, H), lambda i: (0, 0)),
        ],
        out_specs=[
            pl.BlockSpec((1, T, H), lambda i: (i, 0, 0)),
            pl.BlockSpec((1, 1, T), lambda i: (i, 0, 0)),
        ],
        out_shape=[
            jax.ShapeDtypeStruct((B, T, H), jnp.float32),
            jax.ShapeDtypeStruct((B, 1, T), jnp.float32),
        ],
    )(features, mask3, codebook)
    mask_sum = jnp.sum(features_mask, axis=1)
    loss = jnp.sum(sqm[:, 0, :], axis=1) / mask_sum
    return (q, loss, loss)
